# Initial kernel scaffold; baseline (speedup 1.0000x reference)
import jax
import jax.numpy as jnp
from jax.experimental import pallas as pl


def kernel(z, Wg0, bg0, Wt0, bt0, Wg1, bg1, Wt1, bt1, Wg2, bg2, Wt2, bt2, Wout, bout):
    raise NotImplementedError("write your pallas kernel here")


# speedup vs baseline: 2.5346x; 2.5346x over previous
import functools

import jax
import jax.numpy as jnp
import numpy as np
from jax.experimental import pallas as pl
from jax.experimental.pallas import tpu as pltpu


def _path_graph_coeffs(num_joints: int):
    A = np.zeros((num_joints, num_joints), dtype=np.float32)
    for i in range(1, num_joints):
        A[i, i - 1] = 1.0
        A[i - 1, i] = 1.0
    A = A + np.eye(num_joints, dtype=np.float32)
    dinv = 1.0 / np.sqrt(A.sum(axis=1))
    An = dinv[:, None] * A * dinv[None, :]
    c_lo = np.zeros(num_joints, np.float32)
    c_up = np.zeros(num_joints, np.float32)
    c_di = np.diag(An).astype(np.float32).copy()
    c_lo[1:] = An[np.arange(1, num_joints), np.arange(num_joints - 1)]
    c_up[:-1] = An[np.arange(num_joints - 1), np.arange(1, num_joints)]
    return c_lo, c_di, c_up


def _decoder_body(z_ref, clo_ref, cdi_ref, cup_ref,
                  Wg0_ref, bg0_ref, Wt0_ref, bt0_ref,
                  Wg1_ref, bg1_ref, Wt1_ref, bt1_ref,
                  Wg2_ref, bg2_ref, Wt2_ref, bt2_ref,
                  Wout_ref, bout_ref, out_ref, *, num_joints: int):
    h = z_ref[0]
    rows, d = h.shape
    clo = clo_ref[...]
    cdi = cdi_ref[...]
    cup = cup_ref[...]

    blocks = ((Wg0_ref, bg0_ref, Wt0_ref, bt0_ref),
              (Wg1_ref, bg1_ref, Wt1_ref, bt1_ref),
              (Wg2_ref, bg2_ref, Wt2_ref, bt2_ref))
    for Wg_ref, bg_ref, Wt_ref, bt_ref in blocks:
        g = jnp.dot(h, Wg_ref[...], preferred_element_type=jnp.float32)
        zrow = jnp.zeros((1, d), jnp.float32)
        g_dn = jnp.concatenate([zrow, g[:-1, :]], axis=0)
        g_up = jnp.concatenate([g[1:, :], zrow], axis=0)
        m = clo * g_dn + cdi * g + cup * g_up + bg_ref[...]
        h = h + jnp.maximum(m, 0.0)
        ztile = jnp.zeros((num_joints, d), jnp.float32)
        h_dn = jnp.concatenate([ztile, h[:-num_joints, :]], axis=0)
        h_up = jnp.concatenate([h[num_joints:, :], ztile], axis=0)
        y = (jnp.dot(h_dn, Wt_ref[0], preferred_element_type=jnp.float32)
             + jnp.dot(h, Wt_ref[1], preferred_element_type=jnp.float32)
             + jnp.dot(h_up, Wt_ref[2], preferred_element_type=jnp.float32)
             + bt_ref[...])
        h = h + jnp.maximum(y, 0.0)

    out_ref[0] = (jnp.dot(h, Wout_ref[...], preferred_element_type=jnp.float32)
                  + bout_ref[...])


def kernel(z, Wg0, bg0, Wt0, bt0, Wg1, bg1, Wt1, bt1, Wg2, bg2, Wt2, bt2,
           Wout, bout):
    B, T, J, D = z.shape
    TJ = T * J
    zr = z.reshape(B, TJ, D)

    c_lo, c_di, c_up = _path_graph_coeffs(J)
    clo = jnp.asarray(np.tile(c_lo, T)[:, None])
    cdi = jnp.asarray(np.tile(c_di, T)[:, None])
    cup = jnp.asarray(np.tile(c_up, T)[:, None])

    Wt0k = jnp.transpose(Wt0, (2, 1, 0))
    Wt1k = jnp.transpose(Wt1, (2, 1, 0))
    Wt2k = jnp.transpose(Wt2, (2, 1, 0))

    full = lambda shape: pl.BlockSpec(shape, lambda b: (0,) * len(shape))
    wspecs = []
    for Wtk in (Wt0k, Wt1k, Wt2k):
        wspecs += [full((D, D)), full((1, D)), full((3, D, D)), full((1, D))]

    out = pl.pallas_call(
        functools.partial(_decoder_body, num_joints=J),
        grid=(B,),
        in_specs=[pl.BlockSpec((1, TJ, D), lambda b: (b, 0, 0)),
                  full((TJ, 1)), full((TJ, 1)), full((TJ, 1)),
                  *wspecs,
                  full((D, 3)), full((1, 3))],
        out_specs=pl.BlockSpec((1, TJ, 3), lambda b: (b, 0, 0)),
        out_shape=jax.ShapeDtypeStruct((B, TJ, 3), jnp.float32),
        compiler_params=pltpu.CompilerParams(
            dimension_semantics=("parallel",)),
    )(zr, clo, cdi, cup,
      Wg0, bg0.reshape(1, D), Wt0k, bt0.reshape(1, D),
      Wg1, bg1.reshape(1, D), Wt1k, bt1.reshape(1, D),
      Wg2, bg2.reshape(1, D), Wt2k, bt2.reshape(1, D),
      Wout, bout.reshape(1, 3))
    return out.reshape(B, T, J, 3)


# speedup vs baseline: 2.6565x; 1.0481x over previous
import functools

import jax
import jax.numpy as jnp
import numpy as np
from jax.experimental import pallas as pl
from jax.experimental.pallas import tpu as pltpu


def _path_graph_dinv(num_joints: int):
    deg = np.full(num_joints, 3.0, dtype=np.float32)
    deg[0] = deg[-1] = 2.0
    return 1.0 / np.sqrt(deg)


def _decoder_body(z_ref, dv_ref,
                  Wg0_ref, bg0_ref, Wt0_ref, bt0_ref,
                  Wg1_ref, bg1_ref, Wt1_ref, bt1_ref,
                  Wg2_ref, bg2_ref, Wt2_ref, bt2_ref,
                  Wout_ref, bout_ref, out_ref, *, num_joints: int,
                  num_t: int):
    h = z_ref[0]
    rows, d = h.shape
    J = num_joints
    dv = dv_ref[...]

    blocks = ((Wg0_ref, bg0_ref, Wt0_ref, bt0_ref),
              (Wg1_ref, bg1_ref, Wt1_ref, bt1_ref),
              (Wg2_ref, bg2_ref, Wt2_ref, bt2_ref))
    for Wg_ref, bg_ref, Wt_ref, bt_ref in blocks:
        u = (dv * h).astype(jnp.bfloat16)
        u3 = u.reshape(num_t, J, d)
        zslab = jnp.zeros((num_t, 1, d), jnp.bfloat16)
        v3 = (u3
              + jnp.concatenate([zslab, u3[:, :-1, :]], axis=1)
              + jnp.concatenate([u3[:, 1:, :], zslab], axis=1))
        v = v3.reshape(rows, d)
        mv = jnp.dot(v, Wg_ref[...], preferred_element_type=jnp.float32)
        h = h + jnp.maximum(dv * mv + bg_ref[...], 0.0)

        hb = h.astype(jnp.bfloat16)
        ztile = jnp.zeros((J, d), jnp.bfloat16)
        h_dn = jnp.concatenate([ztile, hb[:-J, :]], axis=0)
        h_up = jnp.concatenate([hb[J:, :], ztile], axis=0)
        y = (jnp.dot(h_dn, Wt_ref[0], preferred_element_type=jnp.float32)
             + jnp.dot(hb, Wt_ref[1], preferred_element_type=jnp.float32)
             + jnp.dot(h_up, Wt_ref[2], preferred_element_type=jnp.float32)
             + bt_ref[...])
        h = h + jnp.maximum(y, 0.0)

    out_ref[0] = (jnp.dot(h, Wout_ref[...], preferred_element_type=jnp.float32)
                  + bout_ref[...])


def kernel(z, Wg0, bg0, Wt0, bt0, Wg1, bg1, Wt1, bt1, Wg2, bg2, Wt2, bt2,
           Wout, bout):
    B, T, J, D = z.shape
    TJ = T * J
    zr = z.reshape(B, TJ, D)

    dv = jnp.asarray(np.tile(_path_graph_dinv(J), T)[:, None])

    Wt0k = jnp.transpose(Wt0, (2, 1, 0)).astype(jnp.bfloat16)
    Wt1k = jnp.transpose(Wt1, (2, 1, 0)).astype(jnp.bfloat16)
    Wt2k = jnp.transpose(Wt2, (2, 1, 0)).astype(jnp.bfloat16)

    full = lambda shape: pl.BlockSpec(shape, lambda b: (0,) * len(shape))
    wspecs = []
    for _ in range(3):
        wspecs += [full((D, D)), full((1, D)), full((3, D, D)), full((1, D))]

    out = pl.pallas_call(
        functools.partial(_decoder_body, num_joints=J, num_t=T),
        grid=(B,),
        in_specs=[pl.BlockSpec((1, TJ, D), lambda b: (b, 0, 0)),
                  full((TJ, 1)),
                  *wspecs,
                  full((D, 3)), full((1, 3))],
        out_specs=pl.BlockSpec((1, TJ, 3), lambda b: (b, 0, 0)),
        out_shape=jax.ShapeDtypeStruct((B, TJ, 3), jnp.float32),
        compiler_params=pltpu.CompilerParams(
            dimension_semantics=("parallel",)),
    )(zr, dv,
      Wg0.astype(jnp.bfloat16), bg0.reshape(1, D), Wt0k, bt0.reshape(1, D),
      Wg1.astype(jnp.bfloat16), bg1.reshape(1, D), Wt1k, bt1.reshape(1, D),
      Wg2.astype(jnp.bfloat16), bg2.reshape(1, D), Wt2k, bt2.reshape(1, D),
      Wout, bout.reshape(1, 3))
    return out.reshape(B, T, J, 3)
